# baseline (speedup 1.0000x reference)
import functools
import jax
import jax.numpy as jnp
from jax import lax
from jax.experimental import pallas as pl
from jax.experimental.pallas import tpu as pltpu
from jax.experimental.pallas import tpu_sc as plsc

B, NCH, H, W = 16, 64, 32, 32
NPTS = H * W
BOOK = 1024
N_CLUSTERS = 8
PTILE = 1024
ROWS = PTILE // W

NW = 32
PPW = (B * NPTS) // NW
VL = 16


def _vq_body(c_ref, prec_ref, ze_ref, book_ref, idx_ref, prob_ref, logp_ref,
             bnorm_ref):
    prec = prec_ref[0]
    b_step = pl.program_id(0)
    p_step = pl.program_id(1)
    book = book_ref[0]

    @pl.when(p_step == 0)
    def _():
        bnorm_ref[0, :] = prec * jnp.sum(book * book, axis=1)

    ze_t = ze_ref[0].reshape(NCH, PTILE)
    zep = ze_t.T * (2.0 * prec)
    g = (jnp.dot(zep, book.T, preferred_element_type=jnp.float32)
         - bnorm_ref[0, :][None, :])
    m = jnp.max(g, axis=1, keepdims=True)
    idx = jnp.argmax(g, axis=1)
    sh = g - m
    e = jnp.exp(sh)
    s = jnp.sum(e, axis=1, keepdims=True)
    prob_ref[0] = e * (1.0 / s)
    logp_ref[0] = sh - jnp.log(s)
    idx_ref[0, 0] = idx.astype(jnp.int32) + c_ref[b_step] * BOOK


def _make_sc_gather():
    mesh = plsc.VectorSubcoreMesh(core_axis_name="c", subcore_axis_name="s")

    @functools.partial(
        pl.kernel, mesh=mesh,
        out_type=jax.ShapeDtypeStruct((B, NPTS, 128), jnp.float32),
        scratch_types=[
            pltpu.VMEM((PPW,), jnp.int32),
            pltpu.VMEM((PPW, 128), jnp.float32),
            pltpu.SemaphoreType.DMA,
        ],
    )
    def _gather_k(books_hbm, idx_hbm, out_hbm, idx_v, rows_v, sem):
        wid = lax.axis_index("s") * 2 + lax.axis_index("c")
        b = wid // 2
        half = wid % 2
        pltpu.sync_copy(idx_hbm.at[b, 0, pl.ds(half * PPW, PPW)], idx_v)
        pltpu.async_copy(books_hbm.at[idx_v], rows_v, sem).wait()
        pltpu.sync_copy(rows_v, out_hbm.at[b, pl.ds(half * PPW, PPW)])

    return _gather_k


def kernel(ze, c_logits, books, log_param_q, log_param_q_cls):
    param_q = 1.0 + jnp.exp(log_param_q)
    precision_q = 0.5 / jnp.clip(param_q, 1e-10)
    c = jnp.argmax(c_logits, axis=-1).astype(jnp.int32)
    prec_arr = jnp.reshape(precision_q, (1,)).astype(jnp.float32)

    grid = (B, NPTS // PTILE)
    grid_spec = pltpu.PrefetchScalarGridSpec(
        num_scalar_prefetch=2,
        grid=grid,
        in_specs=[
            pl.BlockSpec((1, NCH, ROWS, W), lambda b, p, c_r, q_r: (b, 0, p, 0)),
            pl.BlockSpec((1, BOOK, NCH), lambda b, p, c_r, q_r: (c_r[b], 0, 0)),
        ],
        out_specs=[
            pl.BlockSpec((1, 1, PTILE), lambda b, p, c_r, q_r: (b, 0, p)),
            pl.BlockSpec((1, PTILE, BOOK), lambda b, p, c_r, q_r: (b, p, 0)),
            pl.BlockSpec((1, PTILE, BOOK), lambda b, p, c_r, q_r: (b, p, 0)),
        ],
        scratch_shapes=[pltpu.VMEM((1, BOOK), jnp.float32)],
    )
    fidx, prob, log_prob = pl.pallas_call(
        _vq_body,
        grid_spec=grid_spec,
        out_shape=[
            jax.ShapeDtypeStruct((B, 1, NPTS), jnp.int32),
            jax.ShapeDtypeStruct((B, NPTS, BOOK), jnp.float32),
            jax.ShapeDtypeStruct((B, NPTS, BOOK), jnp.float32),
        ],
    )(c, prec_arr, ze, books)

    books_dup = jnp.tile(books.reshape(N_CLUSTERS * BOOK, NCH), (1, 2))
    zq_pm = _make_sc_gather()(books_dup, fidx)
    zq = zq_pm[:, :, :NCH].transpose(0, 2, 1).reshape(B, NCH, H, W)
    return (zq, precision_q, prob, log_prob)
